# baseline (speedup 1.0000x reference)
import functools

import jax
import jax.numpy as jnp
from jax import lax
from jax.experimental import pallas as pl
from jax.experimental.pallas import tpu as pltpu
from jax.experimental.pallas import tpu_sc as plsc

DIM = 128
HALF = 64
NXY = 1024
INV_D = 1024.0
N_POINTS = 1024 * 200
NW = 32
CHUNK = 128
PER_W = N_POINTS // NW
N_CHUNKS = PER_W // CHUNK


def _sc_body(t_hbm, w_hbm, b_hbm, table_hbm, out_hbm,
             tbuf, idxA, idxB, gbufA, gbufB, stageA, stageB, wbuf, bbuf,
             semA, semB, wsemA, wsemB):
    wid = lax.axis_index("s") * 2 + lax.axis_index("c")
    base_pt = wid * PER_W

    pltpu.sync_copy(w_hbm, wbuf)
    pltpu.sync_copy(b_hbm, bbuf)
    pltpu.sync_copy(t_hbm.at[pl.ds(base_pt * 2, PER_W * 2)], tbuf)
    w0 = [wbuf[pl.ds(k * 16, 16)] for k in range(4)]
    w1 = [wbuf[pl.ds(64 + k * 16, 16)] for k in range(4)]
    bb = [bbuf[pl.ds(k * 16, 16)] for k in range(4)]
    lanes = lax.broadcasted_iota(jnp.int32, (16,), 0)

    def compute_idx(c, idxbuf):
        for g in range(CHUNK // 16):
            xi = c * (CHUNK * 2) + jnp.int32(g * 32) + 2 * lanes
            xv = plsc.load_gather(tbuf, [xi])
            yv = plsc.load_gather(tbuf, [xi + 1])
            ix = jnp.clip((xv * INV_D).astype(jnp.int32), 0, NXY - 1)
            iy = jnp.clip((yv * INV_D).astype(jnp.int32), 0, NXY - 1)
            idxbuf[pl.ds(g * 16, 16)] = ix * NXY + iy

    def gather_op(idxbuf, gbuf, sem):
        return pltpu.make_async_copy(table_hbm.at[idxbuf], gbuf, sem)

    def write_op(c, stage, wsem):
        return pltpu.make_async_copy(
            stage, out_hbm.at[pl.ds(base_pt + c * CHUNK, CHUNK), :], wsem)

    def lin(c, stage):
        t0 = c * (CHUNK * 2)

        def lin_body(p, _):
            xb = plsc.load_gather(tbuf, [jnp.full((16,), t0 + 2 * p,
                                                  jnp.int32)])
            yb = plsc.load_gather(tbuf, [jnp.full((16,), t0 + 2 * p + 1,
                                                  jnp.int32)])
            for k in range(4):
                stage[p, pl.ds(k * 16, 16)] = xb * w0[k] + yb * w1[k] + bb[k]
            return _

        lax.fori_loop(0, CHUNK, lin_body, None, unroll=2)

    def mv(stage, gbuf):
        def mv_body(p, _):
            for k in range(4):
                stage[p, pl.ds(HALF + k * 16, 16)] = gbuf[p, pl.ds(k * 16, 16)]
            return _

        lax.fori_loop(0, CHUNK, mv_body, None, unroll=2)

    compute_idx(jnp.int32(0), idxA)
    gather_op(idxA, gbufA, semA).start()

    def body(j, _):
        c0 = 2 * j
        c1 = 2 * j + 1
        compute_idx(c1, idxB)
        gather_op(idxB, gbufB, semB).start()

        @pl.when(j > 0)
        def _():
            write_op(c0, stageA, wsemA).wait()

        lin(c0, stageA)
        gather_op(idxA, gbufA, semA).wait()
        mv(stageA, gbufA)
        write_op(c0, stageA, wsemA).start()

        @pl.when(j < N_CHUNKS // 2 - 1)
        def _():
            compute_idx(c0 + 2, idxA)
            gather_op(idxA, gbufA, semA).start()

        @pl.when(j > 0)
        def _():
            write_op(c1, stageB, wsemB).wait()

        lin(c1, stageB)
        gather_op(idxB, gbufB, semB).wait()
        mv(stageB, gbufB)
        write_op(c1, stageB, wsemB).start()
        return _

    lax.fori_loop(0, N_CHUNKS // 2, body, None)
    write_op(jnp.int32(N_CHUNKS - 2), stageA, wsemA).wait()
    write_op(jnp.int32(N_CHUNKS - 1), stageB, wsemB).wait()


PAD_BLK = 4096


def _pad_body(in_ref, out_ref):
    out_ref[:, :HALF] = in_ref[...]
    out_ref[:, HALF:] = jnp.zeros((PAD_BLK, HALF), jnp.float32)


def _pad_table(grid_table):
    return pl.pallas_call(
        _pad_body,
        grid=(NXY * NXY // PAD_BLK,),
        in_specs=[pl.BlockSpec((PAD_BLK, HALF), lambda i: (i, 0))],
        out_specs=pl.BlockSpec((PAD_BLK, DIM), lambda i: (i, 0)),
        out_shape=jax.ShapeDtypeStruct((NXY * NXY, DIM), jnp.float32),
    )(grid_table)


@jax.jit
def kernel(T, W1, b1, grid_table):
    mesh = plsc.VectorSubcoreMesh(core_axis_name="c", subcore_axis_name="s")
    run = pl.kernel(
        _sc_body,
        out_type=jax.ShapeDtypeStruct((N_POINTS, DIM), jnp.float32),
        mesh=mesh,
        scratch_types=[
            pltpu.VMEM((PER_W * 2,), jnp.float32),
            pltpu.VMEM((CHUNK,), jnp.int32),
            pltpu.VMEM((CHUNK,), jnp.int32),
            pltpu.VMEM((CHUNK, HALF), jnp.float32),
            pltpu.VMEM((CHUNK, HALF), jnp.float32),
            pltpu.VMEM((CHUNK, DIM), jnp.float32),
            pltpu.VMEM((CHUNK, DIM), jnp.float32),
            pltpu.VMEM((2 * HALF,), jnp.float32),
            pltpu.VMEM((HALF,), jnp.float32),
            pltpu.SemaphoreType.DMA,
            pltpu.SemaphoreType.DMA,
            pltpu.SemaphoreType.DMA,
            pltpu.SemaphoreType.DMA,
        ],
        compiler_params=pltpu.CompilerParams(
            needs_layout_passes=False, use_tc_tiling_on_sc=False),
    )
    out = run(T.reshape(-1), W1.reshape(-1), b1, grid_table)
    return out.reshape(T.shape[0], T.shape[1], DIM)


# speedup vs baseline: 1.0842x; 1.0842x over previous
import jax
import jax.numpy as jnp
from jax import lax
from jax.experimental import pallas as pl
from jax.experimental.pallas import tpu as pltpu
from jax.experimental.pallas import tpu_sc as plsc

DIM = 128
HALF = 64
NXY = 1024
INV_D = 1024.0
N_POINTS = 1024 * 200
NW = 32
CHUNK = 128
PER_W = N_POINTS // NW
N_CHUNKS = PER_W // CHUNK


def _sc_body(t_hbm, w_hbm, b_hbm, table_hbm, out_hbm,
             tbuf, idxA, idxB, gbufA, gbufB, stageA, stageB, wbuf, bbuf,
             semA, semB, wsemA, wsemB):
    wid = lax.axis_index("s") * 2 + lax.axis_index("c")
    base_pt = wid * PER_W

    pltpu.sync_copy(w_hbm, wbuf)
    pltpu.sync_copy(b_hbm, bbuf)
    pltpu.sync_copy(t_hbm.at[pl.ds(base_pt * 2, PER_W * 2)], tbuf)
    w0 = [wbuf[pl.ds(k * 16, 16)] for k in range(4)]
    w1 = [wbuf[pl.ds(64 + k * 16, 16)] for k in range(4)]
    bb = [bbuf[pl.ds(k * 16, 16)] for k in range(4)]
    lanes = lax.broadcasted_iota(jnp.int32, (16,), 0)

    def compute_idx(c, idxbuf):
        for g in range(CHUNK // 16):
            xi = c * (CHUNK * 2) + jnp.int32(g * 32) + 2 * lanes
            xv = plsc.load_gather(tbuf, [xi])
            yv = plsc.load_gather(tbuf, [xi + 1])
            ix = jnp.clip((xv * INV_D).astype(jnp.int32), 0, NXY - 1)
            iy = jnp.clip((yv * INV_D).astype(jnp.int32), 0, NXY - 1)
            idxbuf[pl.ds(g * 16, 16)] = ix * NXY + iy

    def gather_op(idxbuf, gbuf, sem):
        return pltpu.make_async_copy(table_hbm.at[idxbuf], gbuf, sem)

    def write_op(c, stage, wsem):
        return pltpu.make_async_copy(
            stage, out_hbm.at[pl.ds(base_pt + c * CHUNK, CHUNK), :], wsem)

    def lin(c, stage):
        t0 = c * (CHUNK * 2)

        def lin_body(p, _):
            xb = plsc.load_gather(tbuf, [jnp.full((16,), t0 + 2 * p,
                                                  jnp.int32)])
            yb = plsc.load_gather(tbuf, [jnp.full((16,), t0 + 2 * p + 1,
                                                  jnp.int32)])
            for k in range(4):
                stage[p, pl.ds(k * 16, 16)] = xb * w0[k] + yb * w1[k] + bb[k]
            return _

        lax.fori_loop(0, CHUNK, lin_body, None, unroll=2)

    def mv(stage, gbuf):
        def mv_body(p, _):
            for k in range(4):
                stage[p, pl.ds(HALF + k * 16, 16)] = gbuf[p, pl.ds(k * 16, 16)]
            return _

        lax.fori_loop(0, CHUNK, mv_body, None, unroll=2)

    compute_idx(jnp.int32(0), idxA)
    gather_op(idxA, gbufA, semA).start()

    def body(j, _):
        c0 = 2 * j
        c1 = 2 * j + 1
        compute_idx(c1, idxB)
        gather_op(idxB, gbufB, semB).start()

        @pl.when(j > 0)
        def _():
            write_op(c0, stageA, wsemA).wait()

        lin(c0, stageA)
        gather_op(idxA, gbufA, semA).wait()
        mv(stageA, gbufA)
        write_op(c0, stageA, wsemA).start()

        @pl.when(j < N_CHUNKS // 2 - 1)
        def _():
            compute_idx(c0 + 2, idxA)
            gather_op(idxA, gbufA, semA).start()

        @pl.when(j > 0)
        def _():
            write_op(c1, stageB, wsemB).wait()

        lin(c1, stageB)
        gather_op(idxB, gbufB, semB).wait()
        mv(stageB, gbufB)
        write_op(c1, stageB, wsemB).start()
        return _

    lax.fori_loop(0, N_CHUNKS // 2, body, None)
    write_op(jnp.int32(N_CHUNKS - 2), stageA, wsemA).wait()
    write_op(jnp.int32(N_CHUNKS - 1), stageB, wsemB).wait()


@jax.jit
def kernel(T, W1, b1, grid_table):
    mesh = plsc.VectorSubcoreMesh(core_axis_name="c", subcore_axis_name="s")
    run = pl.kernel(
        _sc_body,
        out_type=jax.ShapeDtypeStruct((N_POINTS, DIM), jnp.float32),
        mesh=mesh,
        scratch_types=[
            pltpu.VMEM((PER_W * 2,), jnp.float32),
            pltpu.VMEM((CHUNK,), jnp.int32),
            pltpu.VMEM((CHUNK,), jnp.int32),
            pltpu.VMEM((CHUNK, DIM), jnp.float32),
            pltpu.VMEM((CHUNK, DIM), jnp.float32),
            pltpu.VMEM((CHUNK, DIM), jnp.float32),
            pltpu.VMEM((CHUNK, DIM), jnp.float32),
            pltpu.VMEM((2 * HALF,), jnp.float32),
            pltpu.VMEM((HALF,), jnp.float32),
            pltpu.SemaphoreType.DMA,
            pltpu.SemaphoreType.DMA,
            pltpu.SemaphoreType.DMA,
            pltpu.SemaphoreType.DMA,
        ],
        compiler_params=pltpu.CompilerParams(
            needs_layout_passes=False, use_tc_tiling_on_sc=True),
    )
    table128 = jnp.pad(grid_table, ((0, 0), (0, DIM - HALF)))
    out = run(T.reshape(-1), W1.reshape(-1), b1, table128)
    return out.reshape(T.shape[0], T.shape[1], DIM)
